# Initial kernel scaffold; baseline (speedup 1.0000x reference)
import jax
import jax.numpy as jnp
from jax.experimental import pallas as pl


def kernel(x):
    raise NotImplementedError("write your pallas kernel here")


# speedup vs baseline: 241.3853x; 241.3853x over previous
import functools

import jax
import jax.numpy as jnp
from jax import lax
from jax.experimental import pallas as pl
from jax.experimental.pallas import tpu as pltpu


def _diag_kernel(x_ref, out_ref, acc_ref, *, T, R, NB, W, ACCW):
    r = pl.program_id(1)

    @pl.when(r == 0)
    def _zero():
        acc_ref[...] = jnp.zeros_like(acc_ref)

    X = x_ref[0]
    row_g = lax.broadcasted_iota(jnp.int32, (R, T), 0) + r * R
    col = lax.broadcasted_iota(jnp.int32, (R, T), 1)
    X = jnp.where((row_g < T - 1) & (col < T - 1), X, 0.0)

    Y = jnp.pad(X, ((0, 0), (0, W - T)))
    G = R
    while G > 1:
        h = G // 2
        Y = pltpu.roll(Y[:h], h, axis=1) + Y[h:]
        G = h

    contrib = jnp.zeros((1, ACCW), Y.dtype)
    for k in range(NB):
        off = T - (R - 1) - R * k
        shifted = jnp.pad(Y, ((0, 0), (off, ACCW - W - off)))
        contrib = contrib + jnp.where(r == k, shifted, 0.0)
    acc_ref[...] = acc_ref[...] + contrib

    @pl.when(r == NB - 1)
    def _finish():
        jd = lax.broadcasted_iota(jnp.int32, (1, T), 1)
        count = (T - 1 - jnp.abs(jd - T // 2)).astype(Y.dtype)
        out_ref[0] = -acc_ref[:, T // 2 : T // 2 + T] / count


@jax.jit
def kernel(x):
    B, T, _ = x.shape
    NB = 4
    R = T // NB
    W = T + R
    ACCW = 2 * T + 128

    out = pl.pallas_call(
        functools.partial(_diag_kernel, T=T, R=R, NB=NB, W=W, ACCW=ACCW),
        grid=(B, NB),
        in_specs=[pl.BlockSpec((1, R, T), lambda b, r: (b, r, 0))],
        out_specs=pl.BlockSpec((1, 1, T), lambda b, r: (b, 0, 0)),
        out_shape=jax.ShapeDtypeStruct((B, 1, T), x.dtype),
        scratch_shapes=[pltpu.VMEM((1, ACCW), jnp.float32)],
        compiler_params=pltpu.CompilerParams(
            dimension_semantics=("arbitrary", "arbitrary"),
        ),
    )(x)
    return out.reshape(B, T)
